# baseline (speedup 1.0000x reference)
import jax
import jax.numpy as jnp
from jax.experimental import pallas as pl

_N = 4096
_H = 128
_TM = 256


def _aggregate(adj_ref, rhs_ref):
    adj = adj_ref[...]
    rhs = rhs_ref[...]
    x = jnp.dot(adj, rhs, preferred_element_type=jnp.float32,
                precision=jax.lax.Precision.HIGHEST)
    return jnp.maximum(x, 0.0)


def _normalize(x):
    n = jnp.sqrt(jnp.sum(x * x, axis=1, keepdims=True))
    return x / jnp.maximum(n, 1e-12)


def _score(x, l1w_ref, l1b_ref, l2w_ref, l2b_ref, l3r_ref, l3b_ref):
    h = jnp.dot(x, l1w_ref[...], preferred_element_type=jnp.float32,
                precision=jax.lax.Precision.HIGHEST) + l1b_ref[...]
    h = jnp.maximum(h, 0.0)
    h = jnp.dot(h, l2w_ref[...], preferred_element_type=jnp.float32,
                precision=jax.lax.Precision.HIGHEST) + l2b_ref[...]
    h = jnp.maximum(h, 0.0)
    return jnp.sum(h * l3r_ref[...], axis=1, keepdims=True) + l3b_ref[...]


def _mid_body(adj_ref, rhs_ref, wn_ref, l1w_ref, l1b_ref, l2w_ref, l2b_ref,
              l3r_ref, l3b_ref, s_in_ref, rhs_next_ref, s_out_ref):
    x = _normalize(_aggregate(adj_ref, rhs_ref))
    s_out_ref[...] = s_in_ref[...] + _score(
        x, l1w_ref, l1b_ref, l2w_ref, l2b_ref, l3r_ref, l3b_ref)
    rhs_next_ref[...] = jnp.dot(x, wn_ref[...],
                                preferred_element_type=jnp.float32,
                                precision=jax.lax.Precision.HIGHEST)


def _last_body(adj_ref, rhs_ref, l1w_ref, l1b_ref, l2w_ref, l2b_ref,
               l3r_ref, l3b_ref, s_in_ref, s_out_ref):
    x = _aggregate(adj_ref, rhs_ref)
    s_out_ref[...] = s_in_ref[...] + _score(
        x, l1w_ref, l1b_ref, l2w_ref, l2b_ref, l3r_ref, l3b_ref)


def _last_mul_body(adj_ref, rhs_ref, l1w_ref, l1b_ref, l2w_ref, l2b_ref,
                   l3r_ref, l3b_ref, s_in_ref, s_other_ref, prod_ref):
    x = _aggregate(adj_ref, rhs_ref)
    s = s_in_ref[...] + _score(
        x, l1w_ref, l1b_ref, l2w_ref, l2b_ref, l3r_ref, l3b_ref)
    prod_ref[...] = s * s_other_ref[...]


def _full(shape):
    return pl.BlockSpec(shape, lambda m: (0, 0))


def _rows(shape):
    return pl.BlockSpec(shape, lambda m: (m, 0))


_GRID = (_N // _TM,)
_ADJ_SPEC = _rows((_TM, _N))
_RHS_SPEC = _full((_N, _H))
_S_SPEC = _rows((_TM, 1))
_F32 = jnp.float32


def _lin_specs():
    return [
        _full((_H, 2 * _H)),
        _full((1, 2 * _H)),
        _full((2 * _H, 2 * _H)),
        _full((1, 2 * _H)),
        _full((1, 2 * _H)),
        _full((1, 1)),
    ]


def _mid_layer(adj, rhs, wn, lins, s_in):
    return pl.pallas_call(
        _mid_body,
        grid=_GRID,
        in_specs=[_ADJ_SPEC, _RHS_SPEC, _full((_H, _H))] + _lin_specs() + [_S_SPEC],
        out_specs=[_rows((_TM, _H)), _S_SPEC],
        out_shape=[jax.ShapeDtypeStruct((_N, _H), _F32),
                   jax.ShapeDtypeStruct((_N, 1), _F32)],
    )(adj, rhs, wn, *lins, s_in)


def _last_layer(adj, rhs, lins, s_in):
    return pl.pallas_call(
        _last_body,
        grid=_GRID,
        in_specs=[_ADJ_SPEC, _RHS_SPEC] + _lin_specs() + [_S_SPEC],
        out_specs=_S_SPEC,
        out_shape=jax.ShapeDtypeStruct((_N, 1), _F32),
    )(adj, rhs, *lins, s_in)


def _last_layer_mul(adj, rhs, lins, s_in, s_other):
    return pl.pallas_call(
        _last_mul_body,
        grid=_GRID,
        in_specs=[_ADJ_SPEC, _RHS_SPEC] + _lin_specs() + [_S_SPEC, _S_SPEC],
        out_specs=_S_SPEC,
        out_shape=jax.ShapeDtypeStruct((_N, 1), _F32),
    )(adj, rhs, *lins, s_in, s_other)


def kernel(adjacent_1, adjacent_2, W1, W2, W3, W4,
           lin1_w, lin1_b, lin2_w, lin2_b, lin3_w, lin3_b):
    lins = (
        lin1_w,
        lin1_b.reshape(1, 2 * _H),
        lin2_w,
        lin2_b.reshape(1, 2 * _H),
        lin3_w.reshape(1, 2 * _H),
        lin3_b.reshape(1, 1),
    )
    zeros = jnp.zeros((_N, 1), _F32)

    def branch_prefix(adj):
        s = zeros
        rhs = W1
        for wn in (W2, W3, W4):
            rhs, s = _mid_layer(adj, rhs, wn, lins, s)
        return rhs, s

    rhs1, s1p = branch_prefix(adjacent_1)
    s1 = _last_layer(adjacent_1, rhs1, lins, s1p)
    rhs2, s2p = branch_prefix(adjacent_2)
    return _last_layer_mul(adjacent_2, rhs2, lins, s2p, s1)


# speedup vs baseline: 2.8850x; 2.8850x over previous
import jax
import jax.numpy as jnp
from jax.experimental import pallas as pl

_N = 4096
_H = 128
_TM = 256
_BF16 = jnp.bfloat16
_F32 = jnp.float32


def _normalize(x):
    n = jnp.sqrt(jnp.sum(x * x, axis=1, keepdims=True))
    return x / jnp.maximum(n, 1e-12)


def _score(x, l1w_ref, l1b_ref, l2w_ref, l2b_ref, l3r_ref, l3b_ref):
    xb = x.astype(_BF16)
    h = jnp.dot(xb, l1w_ref[...].astype(_BF16),
                preferred_element_type=_F32) + l1b_ref[...]
    h = jnp.maximum(h, 0.0)
    h = jnp.dot(h.astype(_BF16), l2w_ref[...].astype(_BF16),
                preferred_element_type=_F32) + l2b_ref[...]
    h = jnp.maximum(h, 0.0)
    return jnp.sum(h * l3r_ref[...], axis=1, keepdims=True) + l3b_ref[...]


def _next_rhs(x, wn_ref):
    r = jnp.dot(x.astype(_BF16), wn_ref[...].astype(_BF16),
                preferred_element_type=_F32)
    return r.astype(_BF16)


def _layer1_body(adj_ref, rhs_ref, wn_ref, l1w_ref, l1b_ref, l2w_ref,
                 l2b_ref, l3r_ref, l3b_ref, s_in_ref,
                 adjb_ref, rhsn_ref, s_out_ref):
    adjb = adj_ref[...].astype(_BF16)
    adjb_ref[...] = adjb
    x = jnp.maximum(jnp.dot(adjb, rhs_ref[...],
                            preferred_element_type=_F32), 0.0)
    x = _normalize(x)
    s_out_ref[...] = s_in_ref[...] + _score(
        x, l1w_ref, l1b_ref, l2w_ref, l2b_ref, l3r_ref, l3b_ref)
    rhsn_ref[...] = _next_rhs(x, wn_ref)


def _mid_body(adjb_ref, rhs_ref, wn_ref, l1w_ref, l1b_ref, l2w_ref,
              l2b_ref, l3r_ref, l3b_ref, s_in_ref, rhsn_ref, s_out_ref):
    x = jnp.maximum(jnp.dot(adjb_ref[...], rhs_ref[...],
                            preferred_element_type=_F32), 0.0)
    x = _normalize(x)
    s_out_ref[...] = s_in_ref[...] + _score(
        x, l1w_ref, l1b_ref, l2w_ref, l2b_ref, l3r_ref, l3b_ref)
    rhsn_ref[...] = _next_rhs(x, wn_ref)


def _last_body(adjb_ref, rhs_ref, l1w_ref, l1b_ref, l2w_ref, l2b_ref,
               l3r_ref, l3b_ref, s_in_ref, s_out_ref):
    x = jnp.maximum(jnp.dot(adjb_ref[...], rhs_ref[...],
                            preferred_element_type=_F32), 0.0)
    s_out_ref[...] = s_in_ref[...] + _score(
        x, l1w_ref, l1b_ref, l2w_ref, l2b_ref, l3r_ref, l3b_ref)


def _last_mul_body(adjb_ref, rhs_ref, l1w_ref, l1b_ref, l2w_ref, l2b_ref,
                   l3r_ref, l3b_ref, s_in_ref, s_other_ref, prod_ref):
    x = jnp.maximum(jnp.dot(adjb_ref[...], rhs_ref[...],
                            preferred_element_type=_F32), 0.0)
    s = s_in_ref[...] + _score(
        x, l1w_ref, l1b_ref, l2w_ref, l2b_ref, l3r_ref, l3b_ref)
    prod_ref[...] = s * s_other_ref[...]


def _cast_body(w_ref, o_ref):
    o_ref[...] = w_ref[...].astype(_BF16)


def _full(shape):
    return pl.BlockSpec(shape, lambda m: (0, 0))


def _rows(shape):
    return pl.BlockSpec(shape, lambda m: (m, 0))


_GRID = (_N // _TM,)
_ADJ_SPEC = _rows((_TM, _N))
_RHS_SPEC = _full((_N, _H))
_S_SPEC = _rows((_TM, 1))


def _lin_specs():
    return [
        _full((_H, 2 * _H)),
        _full((1, 2 * _H)),
        _full((2 * _H, 2 * _H)),
        _full((1, 2 * _H)),
        _full((1, 2 * _H)),
        _full((1, 1)),
    ]


def _cast_bf16(w):
    return pl.pallas_call(
        _cast_body,
        grid=(1,),
        in_specs=[_RHS_SPEC],
        out_specs=_RHS_SPEC,
        out_shape=jax.ShapeDtypeStruct((_N, _H), _BF16),
    )(w)


def _layer1(adj, rhs_b, wn, lins, s_in):
    return pl.pallas_call(
        _layer1_body,
        grid=_GRID,
        in_specs=[_ADJ_SPEC, _RHS_SPEC, _full((_H, _H))] + _lin_specs() + [_S_SPEC],
        out_specs=[_ADJ_SPEC, _rows((_TM, _H)), _S_SPEC],
        out_shape=[jax.ShapeDtypeStruct((_N, _N), _BF16),
                   jax.ShapeDtypeStruct((_N, _H), _BF16),
                   jax.ShapeDtypeStruct((_N, 1), _F32)],
    )(adj, rhs_b, wn, *lins, s_in)


def _mid_layer(adjb, rhs_b, wn, lins, s_in):
    return pl.pallas_call(
        _mid_body,
        grid=_GRID,
        in_specs=[_ADJ_SPEC, _RHS_SPEC, _full((_H, _H))] + _lin_specs() + [_S_SPEC],
        out_specs=[_rows((_TM, _H)), _S_SPEC],
        out_shape=[jax.ShapeDtypeStruct((_N, _H), _BF16),
                   jax.ShapeDtypeStruct((_N, 1), _F32)],
    )(adjb, rhs_b, wn, *lins, s_in)


def _last_layer(adjb, rhs_b, lins, s_in):
    return pl.pallas_call(
        _last_body,
        grid=_GRID,
        in_specs=[_ADJ_SPEC, _RHS_SPEC] + _lin_specs() + [_S_SPEC],
        out_specs=_S_SPEC,
        out_shape=jax.ShapeDtypeStruct((_N, 1), _F32),
    )(adjb, rhs_b, *lins, s_in)


def _last_layer_mul(adjb, rhs_b, lins, s_in, s_other):
    return pl.pallas_call(
        _last_mul_body,
        grid=_GRID,
        in_specs=[_ADJ_SPEC, _RHS_SPEC] + _lin_specs() + [_S_SPEC, _S_SPEC],
        out_specs=_S_SPEC,
        out_shape=jax.ShapeDtypeStruct((_N, 1), _F32),
    )(adjb, rhs_b, *lins, s_in, s_other)


def kernel(adjacent_1, adjacent_2, W1, W2, W3, W4,
           lin1_w, lin1_b, lin2_w, lin2_b, lin3_w, lin3_b):
    lins = (
        lin1_w,
        lin1_b.reshape(1, 2 * _H),
        lin2_w,
        lin2_b.reshape(1, 2 * _H),
        lin3_w.reshape(1, 2 * _H),
        lin3_b.reshape(1, 1),
    )
    zeros = jnp.zeros((_N, 1), _F32)
    w1b = _cast_bf16(W1)

    def branch(adj, extra_s=None):
        adjb, rhs, s = _layer1(adj, w1b, W2, lins, zeros)
        rhs, s = _mid_layer(adjb, rhs, W3, lins, s)
        rhs, s = _mid_layer(adjb, rhs, W4, lins, s)
        if extra_s is None:
            return _last_layer(adjb, rhs, lins, s)
        return _last_layer_mul(adjb, rhs, lins, s, extra_s)

    s1 = branch(adjacent_1)
    return branch(adjacent_2, extra_s=s1)


# speedup vs baseline: 3.5465x; 1.2293x over previous
import jax
import jax.numpy as jnp
from jax.experimental import pallas as pl

_N = 4096
_H = 128
_TM = 512
_MB = _N // _TM
_BF16 = jnp.bfloat16
_F32 = jnp.float32


def _normalize(x):
    n = jnp.sqrt(jnp.sum(x * x, axis=1, keepdims=True))
    return x / jnp.maximum(n, 1e-12)


def _score(x, l1w_ref, l1b_ref, l2w_ref, l2b_ref, l3r_ref, l3b_ref):
    h = jnp.dot(x.astype(_BF16), l1w_ref[...].astype(_BF16),
                preferred_element_type=_F32) + l1b_ref[...]
    h = jnp.maximum(h, 0.0)
    h = jnp.dot(h.astype(_BF16), l2w_ref[...].astype(_BF16),
                preferred_element_type=_F32) + l2b_ref[...]
    h = jnp.maximum(h, 0.0)
    return jnp.sum(h * l3r_ref[...], axis=1, keepdims=True) + l3b_ref[...]


def _next_rhs(x, wn_ref):
    r = jnp.dot(x.astype(_BF16), wn_ref[...].astype(_BF16),
                preferred_element_type=_F32)
    return r.astype(_BF16)


def _layer1_body(adj1_ref, adj2_ref, rhs_ref, wn_ref, l1w_ref, l1b_ref,
                 l2w_ref, l2b_ref, l3r_ref, l3b_ref,
                 adjb_ref, rhsn_ref, s_out_ref):
    b = pl.program_id(0)

    @pl.when(b == 0)
    def _():
        adjb_ref[...] = adj1_ref[...].astype(_BF16)

    @pl.when(b == 1)
    def _():
        adjb_ref[...] = adj2_ref[...].astype(_BF16)

    adjb = adjb_ref[...]
    x = jnp.maximum(jnp.dot(adjb, rhs_ref[...],
                            preferred_element_type=_F32), 0.0)
    x = _normalize(x)
    s_out_ref[...] = _score(
        x, l1w_ref, l1b_ref, l2w_ref, l2b_ref, l3r_ref, l3b_ref)
    rhsn_ref[...] = _next_rhs(x, wn_ref)


def _mid_body(adjb_ref, rhs_ref, wn_ref, l1w_ref, l1b_ref, l2w_ref,
              l2b_ref, l3r_ref, l3b_ref, s_in_ref, rhsn_ref, s_out_ref):
    x = jnp.maximum(jnp.dot(adjb_ref[...], rhs_ref[...],
                            preferred_element_type=_F32), 0.0)
    x = _normalize(x)
    s_out_ref[...] = s_in_ref[...] + _score(
        x, l1w_ref, l1b_ref, l2w_ref, l2b_ref, l3r_ref, l3b_ref)
    rhsn_ref[...] = _next_rhs(x, wn_ref)


def _last_body(adjb_ref, rhs_ref, l1w_ref, l1b_ref, l2w_ref, l2b_ref,
               l3r_ref, l3b_ref, s_in_ref, s_out_ref):
    x = jnp.maximum(jnp.dot(adjb_ref[...], rhs_ref[...],
                            preferred_element_type=_F32), 0.0)
    s_out_ref[...] = s_in_ref[...] + _score(
        x, l1w_ref, l1b_ref, l2w_ref, l2b_ref, l3r_ref, l3b_ref)


def _mul_body(sa_ref, sb_ref, prod_ref):
    prod_ref[...] = sa_ref[...] * sb_ref[...]


def _cast_body(w_ref, o_ref):
    o_ref[...] = w_ref[...].astype(_BF16)


def _full(shape):
    return pl.BlockSpec(shape, lambda b, m: (0, 0))


def _rows2(shape):
    return pl.BlockSpec(shape, lambda b, m: (b * _MB + m, 0))


_GRID = (2, _MB)
_ADJB_SPEC = _rows2((_TM, _N))
_S_SPEC = _rows2((_TM, 1))
_RHS2_SPEC = pl.BlockSpec((_N, _H), lambda b, m: (b, 0))


def _lin_specs():
    return [
        _full((_H, 2 * _H)),
        _full((1, 2 * _H)),
        _full((2 * _H, 2 * _H)),
        _full((1, 2 * _H)),
        _full((1, 2 * _H)),
        _full((1, 1)),
    ]


def _cast_bf16(w):
    return pl.pallas_call(
        _cast_body,
        grid=(1,),
        in_specs=[pl.BlockSpec((_N, _H), lambda m: (0, 0))],
        out_specs=pl.BlockSpec((_N, _H), lambda m: (0, 0)),
        out_shape=jax.ShapeDtypeStruct((_N, _H), _BF16),
    )(w)


def _layer1(adj1, adj2, rhs_b, wn, lins):
    a1_spec = pl.BlockSpec((_TM, _N),
                           lambda b, m: (jnp.where(b == 0, m, _MB - 1), 0))
    a2_spec = pl.BlockSpec((_TM, _N),
                           lambda b, m: (jnp.where(b == 0, 0, m), 0))
    return pl.pallas_call(
        _layer1_body,
        grid=_GRID,
        in_specs=[a1_spec, a2_spec, _full((_N, _H)), _full((_H, _H))]
                 + _lin_specs(),
        out_specs=[_ADJB_SPEC, _rows2((_TM, _H)), _S_SPEC],
        out_shape=[jax.ShapeDtypeStruct((2 * _N, _N), _BF16),
                   jax.ShapeDtypeStruct((2 * _N, _H), _BF16),
                   jax.ShapeDtypeStruct((2 * _N, 1), _F32)],
    )(adj1, adj2, rhs_b, wn, *lins)


def _mid_layer(adjb, rhs_b, wn, lins, s_in):
    return pl.pallas_call(
        _mid_body,
        grid=_GRID,
        in_specs=[_ADJB_SPEC, _RHS2_SPEC, _full((_H, _H))] + _lin_specs()
                 + [_S_SPEC],
        out_specs=[_rows2((_TM, _H)), _S_SPEC],
        out_shape=[jax.ShapeDtypeStruct((2 * _N, _H), _BF16),
                   jax.ShapeDtypeStruct((2 * _N, 1), _F32)],
    )(adjb, rhs_b, wn, *lins, s_in)


def _last_layer(adjb, rhs_b, lins, s_in):
    return pl.pallas_call(
        _last_body,
        grid=_GRID,
        in_specs=[_ADJB_SPEC, _RHS2_SPEC] + _lin_specs() + [_S_SPEC],
        out_specs=_S_SPEC,
        out_shape=jax.ShapeDtypeStruct((2 * _N, 1), _F32),
    )(adjb, rhs_b, *lins, s_in)


def _mul(s_all):
    spec_a = pl.BlockSpec((_TM, 1), lambda m: (m, 0))
    spec_b = pl.BlockSpec((_TM, 1), lambda m: (m + _MB, 0))
    return pl.pallas_call(
        _mul_body,
        grid=(_MB,),
        in_specs=[spec_a, spec_b],
        out_specs=pl.BlockSpec((_TM, 1), lambda m: (m, 0)),
        out_shape=jax.ShapeDtypeStruct((_N, 1), _F32),
    )(s_all, s_all)


def kernel(adjacent_1, adjacent_2, W1, W2, W3, W4,
           lin1_w, lin1_b, lin2_w, lin2_b, lin3_w, lin3_b):
    lins = (
        lin1_w,
        lin1_b.reshape(1, 2 * _H),
        lin2_w,
        lin2_b.reshape(1, 2 * _H),
        lin3_w.reshape(1, 2 * _H),
        lin3_b.reshape(1, 1),
    )
    w1b = _cast_bf16(W1)
    adjb, rhs, s = _layer1(adjacent_1, adjacent_2, w1b, W2, lins)
    rhs, s = _mid_layer(adjb, rhs, W3, lins, s)
    rhs, s = _mid_layer(adjb, rhs, W4, lins, s)
    s = _last_layer(adjb, rhs, lins, s)
    return _mul(s)


# speedup vs baseline: 3.8286x; 1.0795x over previous
import jax
import jax.numpy as jnp
from jax.experimental import pallas as pl
from jax.experimental.pallas import tpu as pltpu

_N = 4096
_H = 128
_TM = 512
_MB = _N // _TM
_KB = 2 * _MB
_BF16 = jnp.bfloat16
_F32 = jnp.float32


def _normalize(x):
    n = jnp.sqrt(jnp.sum(x * x, axis=1, keepdims=True))
    return x / jnp.maximum(n, 1e-12)


def _score(x, l1w_ref, l1b_ref, l2w_ref, l2b_ref, l3r_ref, l3b_ref):
    h = jnp.dot(x.astype(_BF16), l1w_ref[...].astype(_BF16),
                preferred_element_type=_F32) + l1b_ref[...]
    h = jnp.maximum(h, 0.0)
    h = jnp.dot(h.astype(_BF16), l2w_ref[...].astype(_BF16),
                preferred_element_type=_F32) + l2b_ref[...]
    h = jnp.maximum(h, 0.0)
    return jnp.sum(h * l3r_ref[...], axis=1, keepdims=True) + l3b_ref[...]


def _layer1_body(adj1_ref, adj2_ref, w1_ref, wn_ref, l1w_ref, l1b_ref,
                 l2w_ref, l2b_ref, l3r_ref, l3b_ref,
                 adjb_ref, rhsn_ref, s_out_ref):
    b = pl.program_id(0)

    @pl.when(b == 0)
    def _():
        adjb_ref[...] = adj1_ref[...].astype(_BF16)

    @pl.when(b == 1)
    def _():
        adjb_ref[...] = adj2_ref[...].astype(_BF16)

    adjb = adjb_ref[...]
    x = jnp.maximum(jnp.dot(adjb, w1_ref[...].astype(_BF16),
                            preferred_element_type=_F32), 0.0)
    x = _normalize(x)
    s_out_ref[...] = _score(
        x, l1w_ref, l1b_ref, l2w_ref, l2b_ref, l3r_ref, l3b_ref)
    r = jnp.dot(x.astype(_BF16), wn_ref[...].astype(_BF16),
                preferred_element_type=_F32)
    rhsn_ref[...] = r.astype(_BF16)


def _layer1(adj1, adj2, W1, W2, lins):
    a1_spec = pl.BlockSpec((_TM, _N),
                           lambda b, m: (jnp.where(b == 0, m, _MB - 1), 0))
    a2_spec = pl.BlockSpec((_TM, _N),
                           lambda b, m: (jnp.where(b == 0, 0, m), 0))
    full = lambda shape: pl.BlockSpec(shape, lambda b, m: (0, 0))
    rows2 = lambda shape: pl.BlockSpec(shape, lambda b, m: (b * _MB + m, 0))
    return pl.pallas_call(
        _layer1_body,
        grid=(2, _MB),
        in_specs=[a1_spec, a2_spec, full((_N, _H)), full((_H, _H)),
                  full((_H, 2 * _H)), full((1, 2 * _H)),
                  full((2 * _H, 2 * _H)), full((1, 2 * _H)),
                  full((1, 2 * _H)), full((1, 1))],
        out_specs=[rows2((_TM, _N)), rows2((_TM, _H)), rows2((_TM, 1))],
        out_shape=[jax.ShapeDtypeStruct((2 * _N, _N), _BF16),
                   jax.ShapeDtypeStruct((2 * _N, _H), _BF16),
                   jax.ShapeDtypeStruct((2 * _N, 1), _F32)],
    )(adj1, adj2, W1, W2, *lins)


def _rest_body(adjb_ref, rhs0_ref, s1_ref, wn_ref, l1w_ref, l1b_ref,
               l2w_ref, l2b_ref, l3r_ref, l3b_ref,
               prod_ref, x_s, rhs_s, s_s):
    g = pl.program_id(0)
    G = 3 * _KB

    @pl.when(g == 0)
    def _():
        rhs_s[pl.ds(0, _N), :] = rhs0_ref[pl.ds(0, _N), :]

    @pl.when(g == _MB)
    def _():
        rhs_s[pl.ds(_N, _N), :] = rhs0_ref[pl.ds(_N, _N), :]

    @pl.when(g >= 1)
    def _():
        gp = g - 1
        kp = gp % _KB
        lp = gp // _KB
        bp = kp // _MB
        x = x_s[...]

        @pl.when(lp <= 1)
        def _():
            xn = _normalize(x)
            sc = _score(xn, l1w_ref, l1b_ref, l2w_ref, l2b_ref,
                        l3r_ref, l3b_ref)

            @pl.when(lp == 0)
            def _():
                s_s[pl.ds(kp * _TM, _TM), :] = (
                    s1_ref[pl.ds(kp * _TM, _TM), :] + sc)

            @pl.when(lp == 1)
            def _():
                s_s[pl.ds(kp * _TM, _TM), :] = (
                    s_s[pl.ds(kp * _TM, _TM), :] + sc)

            r = jnp.dot(xn.astype(_BF16), wn_ref[0].astype(_BF16),
                        preferred_element_type=_F32)
            woff = (1 - lp % 2) * 2 * _N + bp * _N + (kp % _MB) * _TM
            rhs_s[pl.ds(woff, _TM), :] = r.astype(_BF16)

        @pl.when(lp == 2)
        def _():
            sc = _score(x, l1w_ref, l1b_ref, l2w_ref, l2b_ref,
                        l3r_ref, l3b_ref)
            tot = s_s[pl.ds(kp * _TM, _TM), :] + sc

            @pl.when(bp == 0)
            def _():
                s_s[pl.ds(kp * _TM, _TM), :] = tot

            @pl.when(bp == 1)
            def _():
                prod_ref[...] = tot * s_s[pl.ds((kp - _MB) * _TM, _TM), :]

    @pl.when(g < G)
    def _():
        k = g % _KB
        l = g // _KB
        b = k // _MB
        roff = (l % 2) * 2 * _N + b * _N
        rhs = rhs_s[pl.ds(roff, _N), :]
        x_s[...] = jnp.maximum(
            jnp.dot(adjb_ref[...], rhs, preferred_element_type=_F32), 0.0)


def _rest(adjb, rhs0, s1, wstack, lins):
    G = 3 * _KB
    adjb_spec = pl.BlockSpec(
        (_TM, _N), lambda g: (jnp.where(g < G, g % _KB, _KB - 1), 0))
    rhs0_spec = pl.BlockSpec((2 * _N, _H), lambda g: (0, 0))
    s1_spec = pl.BlockSpec((2 * _N, 1), lambda g: (0, 0))
    wn_spec = pl.BlockSpec(
        (1, _H, _H), lambda g: (jnp.clip((g - 1) // _KB, 0, 1), 0, 0))
    full = lambda shape: pl.BlockSpec(shape, lambda g: (0, 0))
    prod_spec = pl.BlockSpec(
        (_TM, 1), lambda g: (jnp.clip((g - 1) % _KB - _MB, 0, _MB - 1), 0))
    return pl.pallas_call(
        _rest_body,
        grid=(G + 1,),
        in_specs=[adjb_spec, rhs0_spec, s1_spec, wn_spec,
                  full((_H, 2 * _H)), full((1, 2 * _H)),
                  full((2 * _H, 2 * _H)), full((1, 2 * _H)),
                  full((1, 2 * _H)), full((1, 1))],
        out_specs=prod_spec,
        out_shape=jax.ShapeDtypeStruct((_N, 1), _F32),
        scratch_shapes=[
            pltpu.VMEM((_TM, _H), _F32),
            pltpu.VMEM((4 * _N, _H), _BF16),
            pltpu.VMEM((2 * _N, 1), _F32),
        ],
    )(adjb, rhs0, s1, wstack, *lins)


def kernel(adjacent_1, adjacent_2, W1, W2, W3, W4,
           lin1_w, lin1_b, lin2_w, lin2_b, lin3_w, lin3_b):
    lins = (
        lin1_w,
        lin1_b.reshape(1, 2 * _H),
        lin2_w,
        lin2_b.reshape(1, 2 * _H),
        lin3_w.reshape(1, 2 * _H),
        lin3_b.reshape(1, 1),
    )
    adjb, rhs0, s1 = _layer1(adjacent_1, adjacent_2, W1, W2, lins)
    wstack = jnp.stack([W3, W4])
    return _rest(adjb, rhs0, s1, wstack, lins)
